# baseline (speedup 1.0000x reference)
import jax
import jax.numpy as jnp
from jax.experimental import pallas as pl

_B = 8
_N = 2048
_BLK = 256


def _diag_kernel(emb_ref, ns_ref, cov_ref, out_ref):
    ev = jnp.exp(emb_ref[0]) + jnp.exp(ns_ref[0, 0])
    row = jax.lax.broadcasted_iota(jnp.int32, (_BLK, _BLK), 0)
    col = jax.lax.broadcasted_iota(jnp.int32, (_BLK, _BLK), 1)
    out_ref[0] = cov_ref[0] + jnp.where(row == col, ev, 0.0)


def kernel(cov, embeddings, noise_scale):
    emb = embeddings[:, :, -1].reshape(_B, 1, _N)
    ns = noise_scale.reshape(1, 1)
    return pl.pallas_call(
        _diag_kernel,
        grid=(_B, _N // _BLK),
        in_specs=[
            pl.BlockSpec((1, 1, _BLK), lambda b, i: (b, 0, i)),
            pl.BlockSpec((1, 1), lambda b, i: (0, 0)),
            pl.BlockSpec((1, _BLK, _BLK), lambda b, i: (b, i, i)),
        ],
        out_specs=pl.BlockSpec((1, _BLK, _BLK), lambda b, i: (b, i, i)),
        out_shape=jax.ShapeDtypeStruct((_B, _N, _N), jnp.float32),
        input_output_aliases={2: 0},
    )(emb, ns, cov)
